# baseline (speedup 1.0000x reference)
import functools

import jax
import jax.numpy as jnp
from jax import lax
from jax.experimental import pallas as pl
from jax.experimental.pallas import tpu as pltpu
from jax.experimental.pallas import tpu_sc as plsc

N = 10000
E = 320000
C = 128
D_FEAT = 128
EMB = 64
B = 256
NUM_RHS = 100000

NC = 2
NS = 16
NW = NC * NS
EPT = E // NW
K = 80
NCH = EPT // K
NPAD = 10240
RPT = NPAD // NS
ZB = 128
CAP = EPT + K + 16
EPG = EPT // 16
GP = 640
A2 = 2 * B
RT = 2048

_SC_MESH = dict(core_axis_name="c", subcore_axis_name="s")


def _sc_agg_full(h, src3, dstf):
    @functools.partial(
        pl.kernel,
        mesh=plsc.VectorSubcoreMesh(**_SC_MESH),
        out_type=jax.ShapeDtypeStruct((NC * NPAD, C), jnp.float32),
        scratch_types=[
            pltpu.VMEM((NCH, K), jnp.int32),
            pltpu.VMEM((K,), jnp.int32),
            pltpu.VMEM((K,), jnp.int32),
            pltpu.VMEM((K, C), jnp.float32),
            pltpu.VMEM((K, C), jnp.float32),
            pltpu.VMEM_SHARED((NPAD, C), jnp.float32),
            pltpu.SemaphoreType.DMA,
            pltpu.SemaphoreType.DMA,
            pltpu.SemaphoreType.DMA,
            pltpu.SemaphoreType.DMA,
            pltpu.SemaphoreType.DMA,
        ],
    )
    def agg(h_hbm, src_hbm, dst_hbm, out_hbm, src_v, d0, d1, r0, r1,
            acc_sh, semi, sg0, sg1, sd0, sd1):
        c = lax.axis_index("c")
        s = lax.axis_index("s")
        wid = c * NS + s
        row0 = s * RPT
        ebase = wid * EPT

        pltpu.async_copy(src_hbm.at[wid], src_v, semi)

        def zrow(r, carry):
            def zcol(j, carry2):
                r0[r, pl.ds(j * 16, 16)] = jnp.zeros((16,), jnp.float32)
                return carry2
            return lax.fori_loop(0, C // 16, zcol, carry)
        lax.fori_loop(0, K, zrow, 0)
        for t in range(RPT // K):
            pltpu.sync_copy(r0, acc_sh.at[pl.ds(row0 + t * K, K)])

        pltpu.make_async_copy(src_hbm.at[wid], src_v, semi).wait()

        pltpu.async_copy(h_hbm.at[src_v.at[0]], r0, sg0)
        pltpu.async_copy(h_hbm.at[src_v.at[1]], r1, sg1)
        pltpu.async_copy(dst_hbm.at[pl.ds(ebase, K)], d0, sd0)
        pltpu.async_copy(dst_hbm.at[pl.ds(ebase + K, K)], d1, sd1)
        plsc.subcore_barrier()

        def body(io, carry):
            ring = ((r0, sg0, d0, sd0), (r1, sg1, d1, sd1))
            for b, (rb, sgb, db, sdb) in enumerate(ring):
                ch = 2 * io + b
                pltpu.make_async_copy(h_hbm.at[src_v.at[ch]], rb, sgb).wait()
                pltpu.make_async_copy(
                    dst_hbm.at[pl.ds(ebase + ch * K, K)], db, sdb).wait()
                pltpu.sync_copy(rb, acc_sh.at[db], add=True)
                nxt = ch + 2

                @pl.when(nxt < NCH)
                def _():
                    pltpu.async_copy(h_hbm.at[src_v.at[nxt]], rb, sgb)
                    pltpu.async_copy(
                        dst_hbm.at[pl.ds(ebase + nxt * K, K)], db, sdb)
            return carry

        lax.fori_loop(0, NCH // 2, body, 0)
        pltpu.make_async_copy(h_hbm.at[src_v.at[NCH - 1]], r0, sg0).wait()
        pltpu.make_async_copy(
            dst_hbm.at[pl.ds(ebase + (NCH - 1) * K, K)], d0, sd0).wait()
        pltpu.sync_copy(r0, acc_sh.at[d0], add=True)
        plsc.subcore_barrier()

        for t in range(RPT // K):
            pltpu.sync_copy(acc_sh.at[pl.ds(row0 + t * K, K)], r0)
            pltpu.sync_copy(
                r0, out_hbm.at[pl.ds(c * NPAD + row0 + t * K, K)])

    return agg(h, src3, dstf)


def _sc_agg_seed(h, srcf, dredf, goff):
    @functools.partial(
        pl.kernel,
        mesh=plsc.VectorSubcoreMesh(**_SC_MESH),
        out_type=jax.ShapeDtypeStruct((NC * B, C), jnp.float32),
        scratch_types=[
            pltpu.VMEM((EPT,), jnp.int32),
            pltpu.VMEM((EPT,), jnp.int32),
            pltpu.VMEM((GP,), jnp.int32),
            pltpu.VMEM((CAP,), jnp.int32),
            pltpu.VMEM((CAP,), jnp.int32),
            pltpu.VMEM((K,), jnp.int32),
            pltpu.VMEM((K,), jnp.int32),
            pltpu.VMEM((K, C), jnp.float32),
            pltpu.VMEM((K, C), jnp.float32),
            pltpu.VMEM((A2 // NS, C), jnp.float32),
            pltpu.VMEM_SHARED((A2, C), jnp.float32),
            pltpu.SemaphoreType.DMA,
            pltpu.SemaphoreType.DMA,
            pltpu.SemaphoreType.DMA,
        ],
    )
    def agg(h_hbm, src_hbm, dst_hbm, goff_hbm, out_hbm, src_v, dst_v,
            gof_v, csrc, cdst, dc0, dc1, r0, r1, zb_v, acc_sh, semi,
            sg0, sg1):
        c = lax.axis_index("c")
        s = lax.axis_index("s")
        wid = c * NS + s
        ebase = wid * EPT

        pltpu.async_copy(src_hbm.at[pl.ds(ebase, EPT)], src_v, semi)
        pltpu.async_copy(dst_hbm.at[pl.ds(ebase, EPT)], dst_v, semi)
        pltpu.async_copy(goff_hbm.at[wid], gof_v, semi)

        def zrow(r, carry):
            def zcol(j, carry2):
                zb_v[r, pl.ds(j * 16, 16)] = jnp.zeros((16,), jnp.float32)
                return carry2
            return lax.fori_loop(0, C // 16, zcol, carry)
        lax.fori_loop(0, A2 // NS, zrow, 0)
        pltpu.sync_copy(zb_v, acc_sh.at[pl.ds(s * (A2 // NS), A2 // NS)])

        pltpu.make_async_copy(src_hbm.at[pl.ds(ebase, EPT)], src_v,
                              semi).wait()
        pltpu.make_async_copy(dst_hbm.at[pl.ds(ebase, EPT)], dst_v,
                              semi).wait()
        pltpu.make_async_copy(goff_hbm.at[wid], gof_v, semi).wait()

        lane = lax.iota(jnp.int32, 16)

        def crow(blk, carry):
            goffs = gof_v[pl.ds(blk * 16, 16)]
            for j in range(16):
                g16 = blk * 256 + j * 16
                csrc[pl.ds(goffs[j], 16)] = src_v[pl.ds(g16, 16)]
                cdst[pl.ds(goffs[j], 16)] = dst_v[pl.ds(g16, 16)]
            return carry

        lax.fori_loop(0, EPG // 16, crow, 0)
        tail = gof_v[pl.ds(EPG - 1, 16)]
        csrc[pl.ds(tail[0], 16)] = src_v[pl.ds((EPG - 1) * 16, 16)]
        cdst[pl.ds(tail[0], 16)] = dst_v[pl.ds((EPG - 1) * 16, 16)]
        cnt = tail[1]

        for j in range(K // 16 - 1):
            csrc[pl.ds(cnt + j * 16, 16)] = jnp.zeros((16,), jnp.int32)
            cdst[pl.ds(cnt + j * 16, 16)] = B + ((lane + j * 16) & (B - 1))

        @pl.when(0 < cnt)
        def _():
            pltpu.async_copy(h_hbm.at[csrc.at[pl.ds(0, K)]], r0, sg0)

        @pl.when(K < cnt)
        def _():
            pltpu.async_copy(h_hbm.at[csrc.at[pl.ds(K, K)]], r1, sg1)

        plsc.subcore_barrier()

        def body(io, carry):
            ring = ((r0, sg0, dc0), (r1, sg1, dc1))
            for b, (rb, sgb, dcb) in enumerate(ring):
                ch = 2 * io + b

                @pl.when(ch * K < cnt)
                def _():
                    for j in range(K // 16):
                        dcb[pl.ds(j * 16, 16)] = (
                            cdst[pl.ds(ch * K + j * 16, 16)])
                    pltpu.make_async_copy(
                        h_hbm.at[csrc.at[pl.ds(ch * K, K)]], rb, sgb).wait()
                    pltpu.sync_copy(rb, acc_sh.at[dcb], add=True)
                    nxt = ch + 2

                    @pl.when(nxt * K < cnt)
                    def _():
                        pltpu.async_copy(
                            h_hbm.at[csrc.at[pl.ds(nxt * K, K)]], rb, sgb)
            return carry

        lax.fori_loop(0, NCH // 2, body, 0)

        @pl.when((NCH - 1) * K < cnt)
        def _():
            for j in range(K // 16):
                dc0[pl.ds(j * 16, 16)] = (
                    cdst[pl.ds((NCH - 1) * K + j * 16, 16)])
            pltpu.make_async_copy(
                h_hbm.at[csrc.at[pl.ds((NCH - 1) * K, K)]], r0, sg0).wait()
            pltpu.sync_copy(r0, acc_sh.at[dc0], add=True)

        plsc.subcore_barrier()

        pltpu.sync_copy(acc_sh.at[pl.ds(s * 16, 16)], zb_v.at[pl.ds(0, 16)])
        pltpu.sync_copy(zb_v.at[pl.ds(0, 16)],
                        out_hbm.at[pl.ds(c * B + s * 16, 16)])

    return agg(h, srcf, dredf, goff)


def _encoder(x, node_time, seed_time, batch_idx, W_enc, b_enc, id_aware,
             w_time, b_time, dst2, dst3):
    def body(x_ref, nt_ref, st_ref, bi_ref, W_ref, be_ref, ia_ref, wt_ref,
             bt_ref, d2_ref, d3_ref, o_ref, dred_ref, goff_ref):
        h = jnp.dot(x_ref[...], W_ref[...], preferred_element_type=jnp.float32)
        sel = bi_ref[...] == lax.broadcasted_iota(jnp.int32, (N, B), 1)
        st = jnp.sum(jnp.where(sel, st_ref[...], 0.0), axis=1, keepdims=True)
        rel = st - nt_ref[...]
        rowid = lax.broadcasted_iota(jnp.int32, (N, 1), 0)
        h = h + be_ref[...] + jnp.where(rowid < B, 1.0, 0.0) * ia_ref[...]
        o_ref[...] = h + rel * wt_ref[...] + bt_ref[...]

        d2 = d2_ref[...]
        i_flat = lax.broadcasted_iota(jnp.int32, (NW, EPT), 1)
        dred_ref[...] = jnp.where(d2 < B, d2, B + (i_flat & (B - 1)))
        kg = jnp.min(d3_ref[...], axis=2) < B
        kf = jnp.where(kg, 16.0, 0.0).astype(jnp.float32)
        gi = lax.broadcasted_iota(jnp.int32, (EPG, GP), 0)
        gj = lax.broadcasted_iota(jnp.int32, (EPG, GP), 1)
        slt = jnp.where(gi < gj, 1.0, 0.0).astype(jnp.float32)
        goff_ref[...] = jnp.dot(
            kf, slt, preferred_element_type=jnp.float32).astype(jnp.int32)

    return pl.pallas_call(
        body,
        out_shape=[
            jax.ShapeDtypeStruct((N, C), jnp.float32),
            jax.ShapeDtypeStruct((NW, EPT), jnp.int32),
            jax.ShapeDtypeStruct((NW, GP), jnp.int32),
        ],
    )(x, node_time.reshape(N, 1), seed_time.reshape(1, B),
      batch_idx.reshape(N, 1), W_enc, b_enc.reshape(1, C),
      id_aware.reshape(1, C), w_time.reshape(1, C), b_time.reshape(1, C),
      dst2, dst3)


def _sage1(h0, p, W_self1, W_neigh1, b1):
    def body(h_ref, p_ref, ws_ref, wn_ref, b_ref, o_ref):
        agg = p_ref[0:N, :] + p_ref[NPAD:NPAD + N, :]
        o_ref[...] = jax.nn.relu(
            jnp.dot(h_ref[...], ws_ref[...], preferred_element_type=jnp.float32)
            + jnp.dot(agg, wn_ref[...], preferred_element_type=jnp.float32)
            + b_ref[...])

    return pl.pallas_call(
        body,
        out_shape=jax.ShapeDtypeStruct((N, C), jnp.float32),
    )(h0, p, W_self1, W_neigh1, b1.reshape(1, C))


def _score(h1b, a0, a1, W_self2, W_neigh2, b2, lhs_W, lhs_b, rhs_emb):
    def body(h_ref, a0_ref, a1_ref, ws_ref, wn_ref, b_ref, lw_ref, lb_ref,
             r_ref, o_ref, lhs_scr):
        @pl.when(pl.program_id(0) == 0)
        def _():
            agg = a0_ref[...] + a1_ref[...]
            h2 = (jnp.dot(h_ref[...], ws_ref[...],
                          preferred_element_type=jnp.float32)
                  + jnp.dot(agg, wn_ref[...],
                            preferred_element_type=jnp.float32)
                  + b_ref[...])
            lhs_scr[...] = (jnp.dot(h2, lw_ref[...],
                                    preferred_element_type=jnp.float32)
                            + lb_ref[...])

        o_ref[...] = lax.dot_general(
            lhs_scr[...], r_ref[...], (((1,), (1,)), ((), ())),
            preferred_element_type=jnp.float32)

    z = lambda i: (0, 0)
    return pl.pallas_call(
        body,
        grid=(pl.cdiv(NUM_RHS, RT),),
        in_specs=[
            pl.BlockSpec((B, C), z),
            pl.BlockSpec((B, C), z),
            pl.BlockSpec((B, C), z),
            pl.BlockSpec((C, C), z),
            pl.BlockSpec((C, C), z),
            pl.BlockSpec((1, C), z),
            pl.BlockSpec((C, EMB), z),
            pl.BlockSpec((1, EMB), z),
            pl.BlockSpec((RT, EMB), lambda i: (i, 0)),
        ],
        out_specs=pl.BlockSpec((B, RT), lambda i: (0, i)),
        out_shape=jax.ShapeDtypeStruct((B, NUM_RHS), jnp.float32),
        scratch_shapes=[pltpu.VMEM((B, EMB), jnp.float32)],
    )(h1b, a0, a1, W_self2, W_neigh2, b2.reshape(1, C), lhs_W,
      lhs_b.reshape(1, EMB), rhs_emb)


def kernel(x, node_time, seed_time, batch_idx, edge_index, W_enc, b_enc,
           id_aware, w_time, b_time, W_self1, W_neigh1, b1, W_self2,
           W_neigh2, b2, lhs_W, lhs_b, rhs_emb):
    batch_idx = batch_idx.astype(jnp.int32)
    src3 = edge_index[0].reshape(NW, NCH, K)

    h0, dred, goff = _encoder(x, node_time, seed_time, batch_idx, W_enc,
                              b_enc, id_aware, w_time, b_time,
                              edge_index[1].reshape(NW, EPT),
                              edge_index[1].reshape(NW, EPG, 16))
    p1 = _sc_agg_full(h0, src3, edge_index[1])
    h1 = _sage1(h0, p1, W_self1, W_neigh1, b1)
    p2 = _sc_agg_seed(h1, edge_index[0], dred.reshape(E), goff)
    return _score(h1[:B], p2[:B], p2[B:2 * B], W_self2, W_neigh2, b2,
                  lhs_W, lhs_b, rhs_emb)


# speedup vs baseline: 1.0334x; 1.0334x over previous
import functools

import jax
import jax.numpy as jnp
from jax import lax
from jax.experimental import pallas as pl
from jax.experimental.pallas import tpu as pltpu
from jax.experimental.pallas import tpu_sc as plsc

N = 10000
E = 320000
C = 128
D_FEAT = 128
EMB = 64
B = 256
NUM_RHS = 100000

NC = 2
NS = 16
NW = NC * NS
EPT = E // NW
K = 80
NCH = EPT // K
NPAD = 10240
RPT = NPAD // NS
ZB = 128
CAP = EPT + K + 16
EPG = EPT // 16
GP = 640
A2 = 2 * B
RT = 2048

_SC_MESH = dict(core_axis_name="c", subcore_axis_name="s")


def _sc_agg_full(h, src3, dstf):
    @functools.partial(
        pl.kernel,
        mesh=plsc.VectorSubcoreMesh(**_SC_MESH),
        out_type=jax.ShapeDtypeStruct((NC * NPAD, C), jnp.float32),
        scratch_types=[
            pltpu.VMEM((NCH, K), jnp.int32),
            pltpu.VMEM((K,), jnp.int32),
            pltpu.VMEM((K,), jnp.int32),
            pltpu.VMEM((K, C), jnp.float32),
            pltpu.VMEM((K, C), jnp.float32),
            pltpu.VMEM_SHARED((NPAD, C), jnp.float32),
            pltpu.SemaphoreType.DMA,
            pltpu.SemaphoreType.DMA,
            pltpu.SemaphoreType.DMA,
            pltpu.SemaphoreType.DMA,
            pltpu.SemaphoreType.DMA,
        ],
    )
    def agg(h_hbm, src_hbm, dst_hbm, out_hbm, src_v, d0, d1, r0, r1,
            acc_sh, semi, sg0, sg1, sd0, sd1):
        c = lax.axis_index("c")
        s = lax.axis_index("s")
        wid = c * NS + s
        row0 = s * RPT
        ebase = wid * EPT

        pltpu.async_copy(src_hbm.at[wid], src_v, semi)

        def zrow(r, carry):
            def zcol(j, carry2):
                r0[r, pl.ds(j * 16, 16)] = jnp.zeros((16,), jnp.float32)
                return carry2
            return lax.fori_loop(0, C // 16, zcol, carry)
        lax.fori_loop(0, K, zrow, 0)
        for t in range(RPT // K):
            pltpu.sync_copy(r0, acc_sh.at[pl.ds(row0 + t * K, K)])

        pltpu.make_async_copy(src_hbm.at[wid], src_v, semi).wait()

        pltpu.async_copy(h_hbm.at[src_v.at[0]], r0, sg0)
        pltpu.async_copy(h_hbm.at[src_v.at[1]], r1, sg1)
        pltpu.async_copy(dst_hbm.at[pl.ds(ebase, K)], d0, sd0)
        pltpu.async_copy(dst_hbm.at[pl.ds(ebase + K, K)], d1, sd1)
        plsc.subcore_barrier()

        def body(io, carry):
            ring = ((r0, sg0, d0, sd0), (r1, sg1, d1, sd1))
            for b, (rb, sgb, db, sdb) in enumerate(ring):
                ch = 2 * io + b
                pltpu.make_async_copy(h_hbm.at[src_v.at[ch]], rb, sgb).wait()
                pltpu.make_async_copy(
                    dst_hbm.at[pl.ds(ebase + ch * K, K)], db, sdb).wait()
                pltpu.sync_copy(rb, acc_sh.at[db], add=True)
                nxt = ch + 2

                @pl.when(nxt < NCH)
                def _():
                    pltpu.async_copy(h_hbm.at[src_v.at[nxt]], rb, sgb)
                    pltpu.async_copy(
                        dst_hbm.at[pl.ds(ebase + nxt * K, K)], db, sdb)
            return carry

        lax.fori_loop(0, NCH // 2, body, 0)
        pltpu.make_async_copy(h_hbm.at[src_v.at[NCH - 1]], r0, sg0).wait()
        pltpu.make_async_copy(
            dst_hbm.at[pl.ds(ebase + (NCH - 1) * K, K)], d0, sd0).wait()
        pltpu.sync_copy(r0, acc_sh.at[d0], add=True)
        plsc.subcore_barrier()

        for t in range(RPT // K):
            pltpu.sync_copy(acc_sh.at[pl.ds(row0 + t * K, K)], r0)
            pltpu.sync_copy(
                r0, out_hbm.at[pl.ds(c * NPAD + row0 + t * K, K)])

    return agg(h, src3, dstf)


def _edge_prep(dst2, dst3):
    def body(d2_ref, d3_ref, dred_ref, goff_ref):
        d2 = d2_ref[...]
        i_flat = lax.broadcasted_iota(jnp.int32, (NW, EPT), 1)
        dred_ref[...] = jnp.where(d2 < B, d2, B + (i_flat & (B - 1)))
        kg = jnp.min(d3_ref[...], axis=2) < B
        kf = jnp.where(kg, 16.0, 0.0).astype(jnp.float32)
        gi = lax.broadcasted_iota(jnp.int32, (EPG, GP), 0)
        gj = lax.broadcasted_iota(jnp.int32, (EPG, GP), 1)
        slt = jnp.where(gi < gj, 1.0, 0.0).astype(jnp.float32)
        goff_ref[...] = jnp.dot(
            kf, slt, preferred_element_type=jnp.float32).astype(jnp.int32)

    return pl.pallas_call(
        body,
        out_shape=[
            jax.ShapeDtypeStruct((NW, EPT), jnp.int32),
            jax.ShapeDtypeStruct((NW, GP), jnp.int32),
        ],
    )(dst2, dst3)


def _sc_agg_seed(h, srcf, dredf, goff):
    @functools.partial(
        pl.kernel,
        mesh=plsc.VectorSubcoreMesh(**_SC_MESH),
        out_type=jax.ShapeDtypeStruct((NC * B, C), jnp.float32),
        scratch_types=[
            pltpu.VMEM((EPT,), jnp.int32),
            pltpu.VMEM((EPT,), jnp.int32),
            pltpu.VMEM((GP,), jnp.int32),
            pltpu.VMEM((CAP,), jnp.int32),
            pltpu.VMEM((CAP,), jnp.int32),
            pltpu.VMEM((K,), jnp.int32),
            pltpu.VMEM((K,), jnp.int32),
            pltpu.VMEM((K, C), jnp.float32),
            pltpu.VMEM((K, C), jnp.float32),
            pltpu.VMEM((A2 // NS, C), jnp.float32),
            pltpu.VMEM_SHARED((A2, C), jnp.float32),
            pltpu.SemaphoreType.DMA,
            pltpu.SemaphoreType.DMA,
            pltpu.SemaphoreType.DMA,
        ],
    )
    def agg(h_hbm, src_hbm, dst_hbm, goff_hbm, out_hbm, src_v, dst_v,
            gof_v, csrc, cdst, dc0, dc1, r0, r1, zb_v, acc_sh, semi,
            sg0, sg1):
        c = lax.axis_index("c")
        s = lax.axis_index("s")
        wid = c * NS + s
        ebase = wid * EPT

        pltpu.async_copy(src_hbm.at[pl.ds(ebase, EPT)], src_v, semi)
        pltpu.async_copy(dst_hbm.at[pl.ds(ebase, EPT)], dst_v, semi)
        pltpu.async_copy(goff_hbm.at[wid], gof_v, semi)

        def zrow(r, carry):
            def zcol(j, carry2):
                zb_v[r, pl.ds(j * 16, 16)] = jnp.zeros((16,), jnp.float32)
                return carry2
            return lax.fori_loop(0, C // 16, zcol, carry)
        lax.fori_loop(0, A2 // NS, zrow, 0)
        pltpu.sync_copy(zb_v, acc_sh.at[pl.ds(s * (A2 // NS), A2 // NS)])

        pltpu.make_async_copy(src_hbm.at[pl.ds(ebase, EPT)], src_v,
                              semi).wait()
        pltpu.make_async_copy(dst_hbm.at[pl.ds(ebase, EPT)], dst_v,
                              semi).wait()
        pltpu.make_async_copy(goff_hbm.at[wid], gof_v, semi).wait()

        lane = lax.iota(jnp.int32, 16)

        def crow(blk, carry):
            goffs = gof_v[pl.ds(blk * 16, 16)]
            for j in range(16):
                g16 = blk * 256 + j * 16
                csrc[pl.ds(goffs[j], 16)] = src_v[pl.ds(g16, 16)]
                cdst[pl.ds(goffs[j], 16)] = dst_v[pl.ds(g16, 16)]
            return carry

        lax.fori_loop(0, EPG // 16, crow, 0)
        tail = gof_v[pl.ds(EPG - 1, 16)]
        csrc[pl.ds(tail[0], 16)] = src_v[pl.ds((EPG - 1) * 16, 16)]
        cdst[pl.ds(tail[0], 16)] = dst_v[pl.ds((EPG - 1) * 16, 16)]
        cnt = tail[1]

        for j in range(K // 16 - 1):
            csrc[pl.ds(cnt + j * 16, 16)] = jnp.zeros((16,), jnp.int32)
            cdst[pl.ds(cnt + j * 16, 16)] = B + ((lane + j * 16) & (B - 1))

        @pl.when(0 < cnt)
        def _():
            pltpu.async_copy(h_hbm.at[csrc.at[pl.ds(0, K)]], r0, sg0)

        @pl.when(K < cnt)
        def _():
            pltpu.async_copy(h_hbm.at[csrc.at[pl.ds(K, K)]], r1, sg1)

        plsc.subcore_barrier()

        def body(io, carry):
            ring = ((r0, sg0, dc0), (r1, sg1, dc1))
            for b, (rb, sgb, dcb) in enumerate(ring):
                ch = 2 * io + b

                @pl.when(ch * K < cnt)
                def _():
                    for j in range(K // 16):
                        dcb[pl.ds(j * 16, 16)] = (
                            cdst[pl.ds(ch * K + j * 16, 16)])
                    pltpu.make_async_copy(
                        h_hbm.at[csrc.at[pl.ds(ch * K, K)]], rb, sgb).wait()
                    pltpu.sync_copy(rb, acc_sh.at[dcb], add=True)
                    nxt = ch + 2

                    @pl.when(nxt * K < cnt)
                    def _():
                        pltpu.async_copy(
                            h_hbm.at[csrc.at[pl.ds(nxt * K, K)]], rb, sgb)
            return carry

        lax.fori_loop(0, NCH // 2, body, 0)

        @pl.when((NCH - 1) * K < cnt)
        def _():
            for j in range(K // 16):
                dc0[pl.ds(j * 16, 16)] = (
                    cdst[pl.ds((NCH - 1) * K + j * 16, 16)])
            pltpu.make_async_copy(
                h_hbm.at[csrc.at[pl.ds((NCH - 1) * K, K)]], r0, sg0).wait()
            pltpu.sync_copy(r0, acc_sh.at[dc0], add=True)

        plsc.subcore_barrier()

        pltpu.sync_copy(acc_sh.at[pl.ds(s * 16, 16)], zb_v.at[pl.ds(0, 16)])
        pltpu.sync_copy(zb_v.at[pl.ds(0, 16)],
                        out_hbm.at[pl.ds(c * B + s * 16, 16)])

    return agg(h, srcf, dredf, goff)


def _encoder(x, node_time, seed_time, batch_idx, W_enc, b_enc, id_aware,
             w_time, b_time):
    def body(x_ref, nt_ref, st_ref, bi_ref, W_ref, be_ref, ia_ref, wt_ref,
             bt_ref, o_ref):
        h = jnp.dot(x_ref[...], W_ref[...], preferred_element_type=jnp.float32)
        sel = bi_ref[...] == lax.broadcasted_iota(jnp.int32, (N, B), 1)
        st = jnp.sum(jnp.where(sel, st_ref[...], 0.0), axis=1, keepdims=True)
        rel = st - nt_ref[...]
        rowid = lax.broadcasted_iota(jnp.int32, (N, 1), 0)
        h = h + be_ref[...] + jnp.where(rowid < B, 1.0, 0.0) * ia_ref[...]
        o_ref[...] = h + rel * wt_ref[...] + bt_ref[...]

    return pl.pallas_call(
        body,
        out_shape=jax.ShapeDtypeStruct((N, C), jnp.float32),
    )(x, node_time.reshape(N, 1), seed_time.reshape(1, B),
      batch_idx.reshape(N, 1), W_enc, b_enc.reshape(1, C),
      id_aware.reshape(1, C), w_time.reshape(1, C), b_time.reshape(1, C))


def _sage1(h0, p, W_self1, W_neigh1, b1):
    def body(h_ref, p_ref, ws_ref, wn_ref, b_ref, o_ref):
        agg = p_ref[0:N, :] + p_ref[NPAD:NPAD + N, :]
        o_ref[...] = jax.nn.relu(
            jnp.dot(h_ref[...], ws_ref[...], preferred_element_type=jnp.float32)
            + jnp.dot(agg, wn_ref[...], preferred_element_type=jnp.float32)
            + b_ref[...])

    return pl.pallas_call(
        body,
        out_shape=jax.ShapeDtypeStruct((N, C), jnp.float32),
    )(h0, p, W_self1, W_neigh1, b1.reshape(1, C))


def _score(h1b, a0, a1, W_self2, W_neigh2, b2, lhs_W, lhs_b, rhs_emb):
    def body(h_ref, a0_ref, a1_ref, ws_ref, wn_ref, b_ref, lw_ref, lb_ref,
             r_ref, o_ref, lhs_scr):
        @pl.when(pl.program_id(0) == 0)
        def _():
            agg = a0_ref[...] + a1_ref[...]
            h2 = (jnp.dot(h_ref[...], ws_ref[...],
                          preferred_element_type=jnp.float32)
                  + jnp.dot(agg, wn_ref[...],
                            preferred_element_type=jnp.float32)
                  + b_ref[...])
            lhs_scr[...] = (jnp.dot(h2, lw_ref[...],
                                    preferred_element_type=jnp.float32)
                            + lb_ref[...])

        o_ref[...] = lax.dot_general(
            lhs_scr[...], r_ref[...], (((1,), (1,)), ((), ())),
            preferred_element_type=jnp.float32)

    z = lambda i: (0, 0)
    return pl.pallas_call(
        body,
        grid=(pl.cdiv(NUM_RHS, RT),),
        in_specs=[
            pl.BlockSpec((B, C), z),
            pl.BlockSpec((B, C), z),
            pl.BlockSpec((B, C), z),
            pl.BlockSpec((C, C), z),
            pl.BlockSpec((C, C), z),
            pl.BlockSpec((1, C), z),
            pl.BlockSpec((C, EMB), z),
            pl.BlockSpec((1, EMB), z),
            pl.BlockSpec((RT, EMB), lambda i: (i, 0)),
        ],
        out_specs=pl.BlockSpec((B, RT), lambda i: (0, i)),
        out_shape=jax.ShapeDtypeStruct((B, NUM_RHS), jnp.float32),
        scratch_shapes=[pltpu.VMEM((B, EMB), jnp.float32)],
    )(h1b, a0, a1, W_self2, W_neigh2, b2.reshape(1, C), lhs_W,
      lhs_b.reshape(1, EMB), rhs_emb)


def kernel(x, node_time, seed_time, batch_idx, edge_index, W_enc, b_enc,
           id_aware, w_time, b_time, W_self1, W_neigh1, b1, W_self2,
           W_neigh2, b2, lhs_W, lhs_b, rhs_emb):
    batch_idx = batch_idx.astype(jnp.int32)
    src3 = edge_index[0].reshape(NW, NCH, K)

    dred, goff = _edge_prep(edge_index[1].reshape(NW, EPT),
                            edge_index[1].reshape(NW, EPG, 16))
    h0 = _encoder(x, node_time, seed_time, batch_idx, W_enc, b_enc,
                  id_aware, w_time, b_time)
    p1 = _sc_agg_full(h0, src3, edge_index[1])
    h1 = _sage1(h0, p1, W_self1, W_neigh1, b1)
    p2 = _sc_agg_seed(h1, edge_index[0], dred.reshape(E), goff)
    return _score(h1[:B], p2[:B], p2[B:2 * B], W_self2, W_neigh2, b2,
                  lhs_W, lhs_b, rhs_emb)


# speedup vs baseline: 1.0794x; 1.0445x over previous
import functools

import jax
import jax.numpy as jnp
from jax import lax
from jax.experimental import pallas as pl
from jax.experimental.pallas import tpu as pltpu
from jax.experimental.pallas import tpu_sc as plsc

N = 10000
E = 320000
C = 128
D_FEAT = 128
EMB = 64
B = 256
NUM_RHS = 100000

NC = 2
NS = 16
NW = NC * NS
EPT = E // NW
K = 80
NCH = EPT // K
NPAD = 10240
RPT = NPAD // NS
ZB = 128
CAP = EPT + K + 16
EPG = EPT // 16
GP = 640
A2 = 2 * B
RT = 2048

_SC_MESH = dict(core_axis_name="c", subcore_axis_name="s")


def _sc_agg_full(h, srcf, dstf):
    @functools.partial(
        pl.kernel,
        mesh=plsc.VectorSubcoreMesh(**_SC_MESH),
        out_type=jax.ShapeDtypeStruct((NC * NPAD, C), jnp.float32),
        scratch_types=[
            pltpu.VMEM((K,), jnp.int32),
            pltpu.VMEM((K,), jnp.int32),
            pltpu.VMEM((K,), jnp.int32),
            pltpu.VMEM((K,), jnp.int32),
            pltpu.VMEM((K,), jnp.int32),
            pltpu.VMEM((K,), jnp.int32),
            pltpu.VMEM((K, C), jnp.float32),
            pltpu.VMEM((K, C), jnp.float32),
            pltpu.VMEM((K, C), jnp.float32),
            pltpu.VMEM_SHARED((NPAD, C), jnp.float32),
            pltpu.SemaphoreType.DMA,
            pltpu.SemaphoreType.DMA,
            pltpu.SemaphoreType.DMA,
            pltpu.SemaphoreType.DMA,
            pltpu.SemaphoreType.DMA,
            pltpu.SemaphoreType.DMA,
            pltpu.SemaphoreType.DMA,
            pltpu.SemaphoreType.DMA,
            pltpu.SemaphoreType.DMA,
            pltpu.SemaphoreType.DMA,
            pltpu.SemaphoreType.DMA,
            pltpu.SemaphoreType.DMA,
        ],
    )
    def agg(h_hbm, src_hbm, dst_hbm, out_hbm, s0, s1, s2, d0, d1, d2,
            r0, r1, r2, acc_sh,
            si0, si1, si2, sd0, sd1, sd2, sg0, sg1, sg2, ss0, ss1, ss2):
        c = lax.axis_index("c")
        s = lax.axis_index("s")
        wid = c * NS + s
        row0 = s * RPT
        ebase = wid * EPT
        sbufs = (s0, s1, s2)
        dbufs = (d0, d1, d2)
        rbufs = (r0, r1, r2)
        sis = (si0, si1, si2)
        sds = (sd0, sd1, sd2)
        sgs = (sg0, sg1, sg2)
        sss = (ss0, ss1, ss2)

        def src_fetch(ch, b):
            return pltpu.make_async_copy(
                src_hbm.at[pl.ds(ebase + ch * K, K)], sbufs[b], sis[b])

        def dst_fetch(ch, b):
            return pltpu.make_async_copy(
                dst_hbm.at[pl.ds(ebase + ch * K, K)], dbufs[b], sds[b])

        def gather(b):
            return pltpu.make_async_copy(
                h_hbm.at[sbufs[b]], rbufs[b], sgs[b])

        def scatter_start(b):
            pltpu.async_copy(rbufs[b], acc_sh.at[dbufs[b]], sss[b],
                             add=True)

        def scatter_wait(b):
            pltpu.make_async_copy(rbufs[b], acc_sh.at[dbufs[b]],
                                  sss[b]).wait()

        for ch in range(3):
            src_fetch(ch, ch).start()
        for ch in range(2):
            dst_fetch(ch, ch).start()

        def zrow(r, carry):
            def zcol(j, carry2):
                r0[r, pl.ds(j * 16, 16)] = jnp.zeros((16,), jnp.float32)
                return carry2
            return lax.fori_loop(0, C // 16, zcol, carry)
        lax.fori_loop(0, K, zrow, 0)
        for t in range(RPT // K):
            pltpu.sync_copy(r0, acc_sh.at[pl.ds(row0 + t * K, K)])

        src_fetch(0, 0).wait()
        gather(0).start()
        src_fetch(1, 1).wait()
        gather(1).start()
        plsc.subcore_barrier()

        def body(io, carry):
            for b in range(3):
                ch = 3 * io + b
                b2 = (b + 2) % 3
                gather(b).wait()

                @pl.when(ch + 3 < NCH)
                def _():
                    src_fetch(ch + 3, b).start()

                dst_fetch(ch, b).wait()
                scatter_start(b)

                @pl.when(ch + 2 < NCH)
                def _():
                    @pl.when(ch >= 1)
                    def _():
                        scatter_wait(b2)
                    src_fetch(ch + 2, b2).wait()
                    gather(b2).start()
                    dst_fetch(ch + 2, b2).start()
            return carry

        lax.fori_loop(0, NCH // 3, body, 0)
        for ch in (NCH - 2, NCH - 1):
            b = ch % 3
            gather(b).wait()
            dst_fetch(ch, b).wait()
            scatter_start(b)
        for b in ((NCH - 3) % 3, (NCH - 2) % 3, (NCH - 1) % 3):
            scatter_wait(b)
        plsc.subcore_barrier()

        for t in range(RPT // K):
            pltpu.sync_copy(acc_sh.at[pl.ds(row0 + t * K, K)], r0)
            pltpu.sync_copy(
                r0, out_hbm.at[pl.ds(c * NPAD + row0 + t * K, K)])

    return agg(h, srcf, dstf)


def _edge_prep(dst2, dst3):
    def body(d2_ref, d3_ref, dred_ref, goff_ref):
        d2 = d2_ref[...]
        i_flat = lax.broadcasted_iota(jnp.int32, (NW, EPT), 1)
        dred_ref[...] = jnp.where(d2 < B, d2, B + (i_flat & (B - 1)))
        kg = jnp.min(d3_ref[...], axis=2) < B
        kf = jnp.where(kg, 16.0, 0.0).astype(jnp.float32)
        gi = lax.broadcasted_iota(jnp.int32, (EPG, GP), 0)
        gj = lax.broadcasted_iota(jnp.int32, (EPG, GP), 1)
        slt = jnp.where(gi < gj, 1.0, 0.0).astype(jnp.float32)
        goff_ref[...] = jnp.dot(
            kf, slt, preferred_element_type=jnp.float32).astype(jnp.int32)

    return pl.pallas_call(
        body,
        out_shape=[
            jax.ShapeDtypeStruct((NW, EPT), jnp.int32),
            jax.ShapeDtypeStruct((NW, GP), jnp.int32),
        ],
    )(dst2, dst3)


def _sc_agg_seed(h, srcf, dredf, goff):
    @functools.partial(
        pl.kernel,
        mesh=plsc.VectorSubcoreMesh(**_SC_MESH),
        out_type=jax.ShapeDtypeStruct((NC * B, C), jnp.float32),
        scratch_types=[
            pltpu.VMEM((EPT,), jnp.int32),
            pltpu.VMEM((EPT,), jnp.int32),
            pltpu.VMEM((GP,), jnp.int32),
            pltpu.VMEM((CAP,), jnp.int32),
            pltpu.VMEM((CAP,), jnp.int32),
            pltpu.VMEM((K,), jnp.int32),
            pltpu.VMEM((K,), jnp.int32),
            pltpu.VMEM((K, C), jnp.float32),
            pltpu.VMEM((K, C), jnp.float32),
            pltpu.VMEM((A2 // NS, C), jnp.float32),
            pltpu.VMEM_SHARED((A2, C), jnp.float32),
            pltpu.SemaphoreType.DMA,
            pltpu.SemaphoreType.DMA,
            pltpu.SemaphoreType.DMA,
        ],
    )
    def agg(h_hbm, src_hbm, dst_hbm, goff_hbm, out_hbm, src_v, dst_v,
            gof_v, csrc, cdst, dc0, dc1, r0, r1, zb_v, acc_sh, semi,
            sg0, sg1):
        c = lax.axis_index("c")
        s = lax.axis_index("s")
        wid = c * NS + s
        ebase = wid * EPT

        pltpu.async_copy(src_hbm.at[pl.ds(ebase, EPT)], src_v, semi)
        pltpu.async_copy(dst_hbm.at[pl.ds(ebase, EPT)], dst_v, semi)
        pltpu.async_copy(goff_hbm.at[wid], gof_v, semi)

        def zrow(r, carry):
            def zcol(j, carry2):
                zb_v[r, pl.ds(j * 16, 16)] = jnp.zeros((16,), jnp.float32)
                return carry2
            return lax.fori_loop(0, C // 16, zcol, carry)
        lax.fori_loop(0, A2 // NS, zrow, 0)
        pltpu.sync_copy(zb_v, acc_sh.at[pl.ds(s * (A2 // NS), A2 // NS)])

        pltpu.make_async_copy(src_hbm.at[pl.ds(ebase, EPT)], src_v,
                              semi).wait()
        pltpu.make_async_copy(dst_hbm.at[pl.ds(ebase, EPT)], dst_v,
                              semi).wait()
        pltpu.make_async_copy(goff_hbm.at[wid], gof_v, semi).wait()

        lane = lax.iota(jnp.int32, 16)

        def crow(blk, carry):
            goffs = gof_v[pl.ds(blk * 16, 16)]
            for j in range(16):
                g16 = blk * 256 + j * 16
                csrc[pl.ds(goffs[j], 16)] = src_v[pl.ds(g16, 16)]
                cdst[pl.ds(goffs[j], 16)] = dst_v[pl.ds(g16, 16)]
            return carry

        lax.fori_loop(0, EPG // 16, crow, 0)
        tail = gof_v[pl.ds(EPG - 1, 16)]
        csrc[pl.ds(tail[0], 16)] = src_v[pl.ds((EPG - 1) * 16, 16)]
        cdst[pl.ds(tail[0], 16)] = dst_v[pl.ds((EPG - 1) * 16, 16)]
        cnt = tail[1]

        for j in range(K // 16 - 1):
            csrc[pl.ds(cnt + j * 16, 16)] = jnp.zeros((16,), jnp.int32)
            cdst[pl.ds(cnt + j * 16, 16)] = B + ((lane + j * 16) & (B - 1))

        @pl.when(0 < cnt)
        def _():
            pltpu.async_copy(h_hbm.at[csrc.at[pl.ds(0, K)]], r0, sg0)

        @pl.when(K < cnt)
        def _():
            pltpu.async_copy(h_hbm.at[csrc.at[pl.ds(K, K)]], r1, sg1)

        plsc.subcore_barrier()

        def body(io, carry):
            ring = ((r0, sg0, dc0), (r1, sg1, dc1))
            for b, (rb, sgb, dcb) in enumerate(ring):
                ch = 2 * io + b

                @pl.when(ch * K < cnt)
                def _():
                    for j in range(K // 16):
                        dcb[pl.ds(j * 16, 16)] = (
                            cdst[pl.ds(ch * K + j * 16, 16)])
                    pltpu.make_async_copy(
                        h_hbm.at[csrc.at[pl.ds(ch * K, K)]], rb, sgb).wait()
                    pltpu.sync_copy(rb, acc_sh.at[dcb], add=True)
                    nxt = ch + 2

                    @pl.when(nxt * K < cnt)
                    def _():
                        pltpu.async_copy(
                            h_hbm.at[csrc.at[pl.ds(nxt * K, K)]], rb, sgb)
            return carry

        lax.fori_loop(0, NCH // 2, body, 0)

        @pl.when((NCH - 1) * K < cnt)
        def _():
            for j in range(K // 16):
                dc0[pl.ds(j * 16, 16)] = (
                    cdst[pl.ds((NCH - 1) * K + j * 16, 16)])
            pltpu.make_async_copy(
                h_hbm.at[csrc.at[pl.ds((NCH - 1) * K, K)]], r0, sg0).wait()
            pltpu.sync_copy(r0, acc_sh.at[dc0], add=True)

        plsc.subcore_barrier()

        pltpu.sync_copy(acc_sh.at[pl.ds(s * 16, 16)], zb_v.at[pl.ds(0, 16)])
        pltpu.sync_copy(zb_v.at[pl.ds(0, 16)],
                        out_hbm.at[pl.ds(c * B + s * 16, 16)])

    return agg(h, srcf, dredf, goff)


def _encoder(x, node_time, seed_time, batch_idx, W_enc, b_enc, id_aware,
             w_time, b_time):
    def body(x_ref, nt_ref, st_ref, bi_ref, W_ref, be_ref, ia_ref, wt_ref,
             bt_ref, o_ref):
        h = jnp.dot(x_ref[...], W_ref[...], preferred_element_type=jnp.float32)
        sel = bi_ref[...] == lax.broadcasted_iota(jnp.int32, (N, B), 1)
        st = jnp.sum(jnp.where(sel, st_ref[...], 0.0), axis=1, keepdims=True)
        rel = st - nt_ref[...]
        rowid = lax.broadcasted_iota(jnp.int32, (N, 1), 0)
        h = h + be_ref[...] + jnp.where(rowid < B, 1.0, 0.0) * ia_ref[...]
        o_ref[...] = h + rel * wt_ref[...] + bt_ref[...]

    return pl.pallas_call(
        body,
        out_shape=jax.ShapeDtypeStruct((N, C), jnp.float32),
    )(x, node_time.reshape(N, 1), seed_time.reshape(1, B),
      batch_idx.reshape(N, 1), W_enc, b_enc.reshape(1, C),
      id_aware.reshape(1, C), w_time.reshape(1, C), b_time.reshape(1, C))


def _sage1(h0, p, W_self1, W_neigh1, b1):
    def body(h_ref, p_ref, ws_ref, wn_ref, b_ref, o_ref):
        agg = p_ref[0:N, :] + p_ref[NPAD:NPAD + N, :]
        o_ref[...] = jax.nn.relu(
            jnp.dot(h_ref[...], ws_ref[...], preferred_element_type=jnp.float32)
            + jnp.dot(agg, wn_ref[...], preferred_element_type=jnp.float32)
            + b_ref[...])

    return pl.pallas_call(
        body,
        out_shape=jax.ShapeDtypeStruct((N, C), jnp.float32),
    )(h0, p, W_self1, W_neigh1, b1.reshape(1, C))


def _score(h1b, a0, a1, W_self2, W_neigh2, b2, lhs_W, lhs_b, rhs_emb):
    def body(h_ref, a0_ref, a1_ref, ws_ref, wn_ref, b_ref, lw_ref, lb_ref,
             r_ref, o_ref, lhs_scr):
        @pl.when(pl.program_id(0) == 0)
        def _():
            agg = a0_ref[...] + a1_ref[...]
            h2 = (jnp.dot(h_ref[...], ws_ref[...],
                          preferred_element_type=jnp.float32)
                  + jnp.dot(agg, wn_ref[...],
                            preferred_element_type=jnp.float32)
                  + b_ref[...])
            lhs_scr[...] = (jnp.dot(h2, lw_ref[...],
                                    preferred_element_type=jnp.float32)
                            + lb_ref[...])

        o_ref[...] = lax.dot_general(
            lhs_scr[...], r_ref[...], (((1,), (1,)), ((), ())),
            preferred_element_type=jnp.float32)

    z = lambda i: (0, 0)
    return pl.pallas_call(
        body,
        grid=(pl.cdiv(NUM_RHS, RT),),
        in_specs=[
            pl.BlockSpec((B, C), z),
            pl.BlockSpec((B, C), z),
            pl.BlockSpec((B, C), z),
            pl.BlockSpec((C, C), z),
            pl.BlockSpec((C, C), z),
            pl.BlockSpec((1, C), z),
            pl.BlockSpec((C, EMB), z),
            pl.BlockSpec((1, EMB), z),
            pl.BlockSpec((RT, EMB), lambda i: (i, 0)),
        ],
        out_specs=pl.BlockSpec((B, RT), lambda i: (0, i)),
        out_shape=jax.ShapeDtypeStruct((B, NUM_RHS), jnp.float32),
        scratch_shapes=[pltpu.VMEM((B, EMB), jnp.float32)],
    )(h1b, a0, a1, W_self2, W_neigh2, b2.reshape(1, C), lhs_W,
      lhs_b.reshape(1, EMB), rhs_emb)


def kernel(x, node_time, seed_time, batch_idx, edge_index, W_enc, b_enc,
           id_aware, w_time, b_time, W_self1, W_neigh1, b1, W_self2,
           W_neigh2, b2, lhs_W, lhs_b, rhs_emb):
    batch_idx = batch_idx.astype(jnp.int32)

    dred, goff = _edge_prep(edge_index[1].reshape(NW, EPT),
                            edge_index[1].reshape(NW, EPG, 16))
    h0 = _encoder(x, node_time, seed_time, batch_idx, W_enc, b_enc,
                  id_aware, w_time, b_time)
    p1 = _sc_agg_full(h0, edge_index[0], edge_index[1])
    h1 = _sage1(h0, p1, W_self1, W_neigh1, b1)
    p2 = _sc_agg_seed(h1, edge_index[0], dred.reshape(E), goff)
    return _score(h1[:B], p2[:B], p2[B:2 * B], W_self2, W_neigh2, b2,
                  lhs_W, lhs_b, rhs_emb)


# speedup vs baseline: 1.0806x; 1.0011x over previous
import functools

import jax
import jax.numpy as jnp
from jax import lax
from jax.experimental import pallas as pl
from jax.experimental.pallas import tpu as pltpu
from jax.experimental.pallas import tpu_sc as plsc

N = 10000
E = 320000
C = 128
D_FEAT = 128
EMB = 64
B = 256
NUM_RHS = 100000

NC = 2
NS = 16
NW = NC * NS
EPT = E // NW
K = 80
NCH = EPT // K
NPAD = 10240
RPT = NPAD // NS
CAP = EPT + K + 16
EPG = EPT // 16
GP = 640
A2 = 2 * B
RT = 2048

_SC_MESH = dict(core_axis_name="c", subcore_axis_name="s")


def _sc_agg_full(h, srcf, dstf):
    @functools.partial(
        pl.kernel,
        mesh=plsc.VectorSubcoreMesh(**_SC_MESH),
        out_type=jax.ShapeDtypeStruct((NC * NPAD, C), jnp.float32),
        scratch_types=[
            pltpu.VMEM((K,), jnp.int32),
            pltpu.VMEM((K,), jnp.int32),
            pltpu.VMEM((K,), jnp.int32),
            pltpu.VMEM((K,), jnp.int32),
            pltpu.VMEM((K,), jnp.int32),
            pltpu.VMEM((K,), jnp.int32),
            pltpu.VMEM((K, C), jnp.float32),
            pltpu.VMEM((K, C), jnp.float32),
            pltpu.VMEM((K, C), jnp.float32),
            pltpu.VMEM_SHARED((NPAD, C), jnp.float32),
            pltpu.SemaphoreType.DMA,
            pltpu.SemaphoreType.DMA,
            pltpu.SemaphoreType.DMA,
            pltpu.SemaphoreType.DMA,
            pltpu.SemaphoreType.DMA,
            pltpu.SemaphoreType.DMA,
            pltpu.SemaphoreType.DMA,
            pltpu.SemaphoreType.DMA,
            pltpu.SemaphoreType.DMA,
            pltpu.SemaphoreType.DMA,
            pltpu.SemaphoreType.DMA,
            pltpu.SemaphoreType.DMA,
        ],
    )
    def agg(h_hbm, src_hbm, dst_hbm, out_hbm, s0, s1, s2, d0, d1, d2,
            r0, r1, r2, acc_sh,
            si0, si1, si2, sd0, sd1, sd2, sg0, sg1, sg2, ss0, ss1, ss2):
        c = lax.axis_index("c")
        s = lax.axis_index("s")
        wid = c * NS + s
        row0 = s * RPT
        ebase = wid * EPT
        sbufs = (s0, s1, s2)
        dbufs = (d0, d1, d2)
        rbufs = (r0, r1, r2)
        sis = (si0, si1, si2)
        sds = (sd0, sd1, sd2)
        sgs = (sg0, sg1, sg2)
        sss = (ss0, ss1, ss2)

        def src_fetch(ch, b):
            return pltpu.make_async_copy(
                src_hbm.at[pl.ds(ebase + ch * K, K)], sbufs[b], sis[b])

        def dst_fetch(ch, b):
            return pltpu.make_async_copy(
                dst_hbm.at[pl.ds(ebase + ch * K, K)], dbufs[b], sds[b])

        def gather(b):
            return pltpu.make_async_copy(
                h_hbm.at[sbufs[b]], rbufs[b], sgs[b])

        def scatter_start(b):
            pltpu.async_copy(rbufs[b], acc_sh.at[dbufs[b]], sss[b],
                             add=True)

        def scatter_wait(b):
            pltpu.make_async_copy(rbufs[b], acc_sh.at[dbufs[b]],
                                  sss[b]).wait()

        for ch in range(3):
            src_fetch(ch, ch).start()
        for ch in range(2):
            dst_fetch(ch, ch).start()

        def zrow(r, carry):
            def zcol(j, carry2):
                r0[r, pl.ds(j * 16, 16)] = jnp.zeros((16,), jnp.float32)
                return carry2
            return lax.fori_loop(0, C // 16, zcol, carry)
        lax.fori_loop(0, K, zrow, 0)
        for t in range(RPT // K):
            pltpu.sync_copy(r0, acc_sh.at[pl.ds(row0 + t * K, K)])

        src_fetch(0, 0).wait()
        gather(0).start()
        src_fetch(1, 1).wait()
        gather(1).start()
        plsc.subcore_barrier()

        def body(io, carry):
            for b in range(3):
                ch = 3 * io + b
                b2 = (b + 2) % 3
                gather(b).wait()

                @pl.when(ch + 3 < NCH)
                def _():
                    src_fetch(ch + 3, b).start()

                dst_fetch(ch, b).wait()
                scatter_start(b)

                @pl.when(ch + 2 < NCH)
                def _():
                    @pl.when(ch >= 1)
                    def _():
                        scatter_wait(b2)
                    src_fetch(ch + 2, b2).wait()
                    gather(b2).start()
                    dst_fetch(ch + 2, b2).start()
            return carry

        lax.fori_loop(0, NCH // 3, body, 0)
        for ch in (NCH - 2, NCH - 1):
            b = ch % 3
            gather(b).wait()
            dst_fetch(ch, b).wait()
            scatter_start(b)
        for b in ((NCH - 3) % 3, (NCH - 2) % 3, (NCH - 1) % 3):
            scatter_wait(b)
        plsc.subcore_barrier()

        for t in range(RPT // K):
            pltpu.sync_copy(acc_sh.at[pl.ds(row0 + t * K, K)], r0)
            pltpu.sync_copy(
                r0, out_hbm.at[pl.ds(c * NPAD + row0 + t * K, K)])

    return agg(h, srcf, dstf)


def _edge_prep(dst2, dst3):
    def body(d2_ref, d3_ref, dred_ref, goff_ref):
        d2 = d2_ref[...]
        i_flat = lax.broadcasted_iota(jnp.int32, (NW, EPT), 1)
        dred_ref[...] = jnp.where(d2 < B, d2, B + (i_flat & (B - 1)))
        kg = jnp.min(d3_ref[...], axis=2) < B
        kf = jnp.where(kg, 16.0, 0.0).astype(jnp.float32)
        gi = lax.broadcasted_iota(jnp.int32, (EPG, GP), 0)
        gj = lax.broadcasted_iota(jnp.int32, (EPG, GP), 1)
        slt = jnp.where(gi < gj, 1.0, 0.0).astype(jnp.float32)
        goff_ref[...] = jnp.dot(
            kf, slt, preferred_element_type=jnp.float32).astype(jnp.int32)

    return pl.pallas_call(
        body,
        out_shape=[
            jax.ShapeDtypeStruct((NW, EPT), jnp.int32),
            jax.ShapeDtypeStruct((NW, GP), jnp.int32),
        ],
    )(dst2, dst3)


def _sc_agg_seed(h, srcf, dredf, goff):
    @functools.partial(
        pl.kernel,
        mesh=plsc.VectorSubcoreMesh(**_SC_MESH),
        out_type=jax.ShapeDtypeStruct((NC * B, C), jnp.float32),
        scratch_types=[
            pltpu.VMEM((EPT,), jnp.int32),
            pltpu.VMEM((EPT,), jnp.int32),
            pltpu.VMEM((GP,), jnp.int32),
            pltpu.VMEM((CAP,), jnp.int32),
            pltpu.VMEM((CAP,), jnp.int32),
            pltpu.VMEM((K,), jnp.int32),
            pltpu.VMEM((K,), jnp.int32),
            pltpu.VMEM((K, C), jnp.float32),
            pltpu.VMEM((K, C), jnp.float32),
            pltpu.VMEM((A2 // NS, C), jnp.float32),
            pltpu.VMEM_SHARED((A2, C), jnp.float32),
            pltpu.SemaphoreType.DMA,
            pltpu.SemaphoreType.DMA,
            pltpu.SemaphoreType.DMA,
        ],
    )
    def agg(h_hbm, src_hbm, dst_hbm, goff_hbm, out_hbm, src_v, dst_v,
            gof_v, csrc, cdst, dc0, dc1, r0, r1, zb_v, acc_sh, semi,
            sg0, sg1):
        c = lax.axis_index("c")
        s = lax.axis_index("s")
        wid = c * NS + s
        ebase = wid * EPT

        pltpu.async_copy(src_hbm.at[pl.ds(ebase, EPT)], src_v, semi)
        pltpu.async_copy(dst_hbm.at[pl.ds(ebase, EPT)], dst_v, semi)
        pltpu.async_copy(goff_hbm.at[wid], gof_v, semi)

        def zrow(r, carry):
            def zcol(j, carry2):
                zb_v[r, pl.ds(j * 16, 16)] = jnp.zeros((16,), jnp.float32)
                return carry2
            return lax.fori_loop(0, C // 16, zcol, carry)
        lax.fori_loop(0, A2 // NS, zrow, 0)
        pltpu.sync_copy(zb_v, acc_sh.at[pl.ds(s * (A2 // NS), A2 // NS)])

        pltpu.make_async_copy(src_hbm.at[pl.ds(ebase, EPT)], src_v,
                              semi).wait()
        pltpu.make_async_copy(dst_hbm.at[pl.ds(ebase, EPT)], dst_v,
                              semi).wait()
        pltpu.make_async_copy(goff_hbm.at[wid], gof_v, semi).wait()

        lane = lax.iota(jnp.int32, 16)

        def crow(blk, carry):
            goffs = gof_v[pl.ds(blk * 16, 16)]
            for j in range(16):
                g16 = blk * 256 + j * 16
                csrc[pl.ds(goffs[j], 16)] = src_v[pl.ds(g16, 16)]
                cdst[pl.ds(goffs[j], 16)] = dst_v[pl.ds(g16, 16)]
            return carry

        lax.fori_loop(0, EPG // 16, crow, 0)
        tail = gof_v[pl.ds(EPG - 1, 16)]
        csrc[pl.ds(tail[0], 16)] = src_v[pl.ds((EPG - 1) * 16, 16)]
        cdst[pl.ds(tail[0], 16)] = dst_v[pl.ds((EPG - 1) * 16, 16)]
        cnt = tail[1]

        for j in range(K // 16 - 1):
            csrc[pl.ds(cnt + j * 16, 16)] = jnp.zeros((16,), jnp.int32)
            cdst[pl.ds(cnt + j * 16, 16)] = B + ((lane + j * 16) & (B - 1))

        @pl.when(0 < cnt)
        def _():
            pltpu.async_copy(h_hbm.at[csrc.at[pl.ds(0, K)]], r0, sg0)

        @pl.when(K < cnt)
        def _():
            pltpu.async_copy(h_hbm.at[csrc.at[pl.ds(K, K)]], r1, sg1)

        plsc.subcore_barrier()

        def body(io, carry):
            ring = ((r0, sg0, dc0), (r1, sg1, dc1))
            for b, (rb, sgb, dcb) in enumerate(ring):
                ch = 2 * io + b

                @pl.when(ch * K < cnt)
                def _():
                    for j in range(K // 16):
                        dcb[pl.ds(j * 16, 16)] = (
                            cdst[pl.ds(ch * K + j * 16, 16)])
                    pltpu.make_async_copy(
                        h_hbm.at[csrc.at[pl.ds(ch * K, K)]], rb, sgb).wait()
                    pltpu.sync_copy(rb, acc_sh.at[dcb], add=True)
                    nxt = ch + 2

                    @pl.when(nxt * K < cnt)
                    def _():
                        pltpu.async_copy(
                            h_hbm.at[csrc.at[pl.ds(nxt * K, K)]], rb, sgb)
            return carry

        lax.fori_loop(0, NCH // 2, body, 0)

        @pl.when((NCH - 1) * K < cnt)
        def _():
            for j in range(K // 16):
                dc0[pl.ds(j * 16, 16)] = (
                    cdst[pl.ds((NCH - 1) * K + j * 16, 16)])
            pltpu.make_async_copy(
                h_hbm.at[csrc.at[pl.ds((NCH - 1) * K, K)]], r0, sg0).wait()
            pltpu.sync_copy(r0, acc_sh.at[dc0], add=True)

        plsc.subcore_barrier()

        pltpu.sync_copy(acc_sh.at[pl.ds(s * 16, 16)], zb_v.at[pl.ds(0, 16)])
        pltpu.sync_copy(zb_v.at[pl.ds(0, 16)],
                        out_hbm.at[pl.ds(c * B + s * 16, 16)])

    return agg(h, srcf, dredf, goff)


def _encoder(x, node_time, seed_time, batch_idx, W_enc, b_enc, id_aware,
             w_time, b_time):
    def body(x_ref, nt_ref, st_ref, bi_ref, W_ref, be_ref, ia_ref, wt_ref,
             bt_ref, o_ref):
        h = jnp.dot(x_ref[...], W_ref[...], preferred_element_type=jnp.float32)
        sel = bi_ref[...] == lax.broadcasted_iota(jnp.int32, (N, B), 1)
        st = jnp.sum(jnp.where(sel, st_ref[...], 0.0), axis=1, keepdims=True)
        rel = st - nt_ref[...]
        rowid = lax.broadcasted_iota(jnp.int32, (N, 1), 0)
        h = h + be_ref[...] + jnp.where(rowid < B, 1.0, 0.0) * ia_ref[...]
        o_ref[...] = h + rel * wt_ref[...] + bt_ref[...]

    return pl.pallas_call(
        body,
        out_shape=jax.ShapeDtypeStruct((N, C), jnp.float32),
    )(x, node_time.reshape(N, 1), seed_time.reshape(1, B),
      batch_idx.reshape(N, 1), W_enc, b_enc.reshape(1, C),
      id_aware.reshape(1, C), w_time.reshape(1, C), b_time.reshape(1, C))


def _sage1(h0, p, W_self1, W_neigh1, b1):
    def body(h_ref, p_ref, ws_ref, wn_ref, b_ref, o_ref):
        agg = p_ref[0:N, :] + p_ref[NPAD:NPAD + N, :]
        o_ref[...] = jax.nn.relu(
            jnp.dot(h_ref[...], ws_ref[...], preferred_element_type=jnp.float32)
            + jnp.dot(agg, wn_ref[...], preferred_element_type=jnp.float32)
            + b_ref[...])

    return pl.pallas_call(
        body,
        out_shape=jax.ShapeDtypeStruct((N, C), jnp.float32),
    )(h0, p, W_self1, W_neigh1, b1.reshape(1, C))


def _score(h1b, a0, a1, W_self2, W_neigh2, b2, lhs_W, lhs_b, rhs_emb):
    def body(h_ref, a0_ref, a1_ref, ws_ref, wn_ref, b_ref, lw_ref, lb_ref,
             r_ref, o_ref, lhs_scr):
        @pl.when(pl.program_id(0) == 0)
        def _():
            agg = a0_ref[...] + a1_ref[...]
            h2 = (jnp.dot(h_ref[...], ws_ref[...],
                          preferred_element_type=jnp.float32)
                  + jnp.dot(agg, wn_ref[...],
                            preferred_element_type=jnp.float32)
                  + b_ref[...])
            lhs_scr[...] = (jnp.dot(h2, lw_ref[...],
                                    preferred_element_type=jnp.float32)
                            + lb_ref[...])

        o_ref[...] = lax.dot_general(
            lhs_scr[...], r_ref[...], (((1,), (1,)), ((), ())),
            preferred_element_type=jnp.float32)

    z = lambda i: (0, 0)
    return pl.pallas_call(
        body,
        grid=(pl.cdiv(NUM_RHS, RT),),
        in_specs=[
            pl.BlockSpec((B, C), z),
            pl.BlockSpec((B, C), z),
            pl.BlockSpec((B, C), z),
            pl.BlockSpec((C, C), z),
            pl.BlockSpec((C, C), z),
            pl.BlockSpec((1, C), z),
            pl.BlockSpec((C, EMB), z),
            pl.BlockSpec((1, EMB), z),
            pl.BlockSpec((RT, EMB), lambda i: (i, 0)),
        ],
        out_specs=pl.BlockSpec((B, RT), lambda i: (0, i)),
        out_shape=jax.ShapeDtypeStruct((B, NUM_RHS), jnp.float32),
        scratch_shapes=[pltpu.VMEM((B, EMB), jnp.float32)],
    )(h1b, a0, a1, W_self2, W_neigh2, b2.reshape(1, C), lhs_W,
      lhs_b.reshape(1, EMB), rhs_emb)


def kernel(x, node_time, seed_time, batch_idx, edge_index, W_enc, b_enc,
           id_aware, w_time, b_time, W_self1, W_neigh1, b1, W_self2,
           W_neigh2, b2, lhs_W, lhs_b, rhs_emb):
    batch_idx = batch_idx.astype(jnp.int32)

    dred, goff = _edge_prep(edge_index[1].reshape(NW, EPT),
                            edge_index[1].reshape(NW, EPG, 16))
    h0 = _encoder(x, node_time, seed_time, batch_idx, W_enc, b_enc,
                  id_aware, w_time, b_time)
    p1 = _sc_agg_full(h0, edge_index[0], edge_index[1])
    h1 = _sage1(h0, p1, W_self1, W_neigh1, b1)
    p2 = _sc_agg_seed(h1, edge_index[0], dred.reshape(E), goff)
    return _score(h1[:B], p2[:B], p2[B:2 * B], W_self2, W_neigh2, b2,
                  lhs_W, lhs_b, rhs_emb)
